# baseline (speedup 1.0000x reference)
import jax
import jax.numpy as jnp
from jax.experimental import pallas as pl
from jax.experimental.pallas import tpu as pltpu

ROWS, D = 8192, 768
N_DMA = 8
ROWS_PER_DMA = ROWS // N_DMA


def _body(w_ref, o_ref, sem):
    copies = [
        pltpu.make_async_copy(
            w_ref.at[pl.ds(i * ROWS_PER_DMA, ROWS_PER_DMA)],
            o_ref.at[pl.ds(i * ROWS_PER_DMA, ROWS_PER_DMA)],
            sem.at[i],
        )
        for i in range(N_DMA)
    ]
    for c in copies:
        c.start()
    for c in copies:
        c.wait()


def kernel(x, W):
    del x
    return pl.pallas_call(
        _body,
        in_specs=[pl.BlockSpec(memory_space=pl.ANY)],
        out_specs=pl.BlockSpec(memory_space=pl.ANY),
        out_shape=jax.ShapeDtypeStruct((ROWS, D), jnp.float32),
        scratch_shapes=[pltpu.SemaphoreType.DMA((N_DMA,))],
    )(W)


# speedup vs baseline: 43.0047x; 43.0047x over previous
import jax
import jax.numpy as jnp
from jax.experimental import pallas as pl

ROWS, D = 8192, 768
BLOCK_ROWS = 1024


def _copy_body(w_ref, o_ref):
    o_ref[...] = w_ref[...]


def kernel(x, W):
    del x
    return pl.pallas_call(
        _copy_body,
        grid=(ROWS // BLOCK_ROWS,),
        in_specs=[pl.BlockSpec((BLOCK_ROWS, D), lambda i: (i, 0))],
        out_specs=pl.BlockSpec((BLOCK_ROWS, D), lambda i: (i, 0)),
        out_shape=jax.ShapeDtypeStruct((ROWS, D), jnp.float32),
    )(W)


# speedup vs baseline: 46.5455x; 1.0823x over previous
import jax
import jax.numpy as jnp
from jax.experimental import pallas as pl

ROWS, D = 8192, 768
BLOCK_ROWS = 2048


def _copy_body(w_ref, o_ref):
    o_ref[...] = w_ref[...]


def kernel(x, W):
    del x
    return pl.pallas_call(
        _copy_body,
        grid=(ROWS // BLOCK_ROWS,),
        in_specs=[pl.BlockSpec((BLOCK_ROWS, D), lambda i: (i, 0))],
        out_specs=pl.BlockSpec((BLOCK_ROWS, D), lambda i: (i, 0)),
        out_shape=jax.ShapeDtypeStruct((ROWS, D), jnp.float32),
    )(W)


# speedup vs baseline: 49.3879x; 1.0611x over previous
import jax
import jax.numpy as jnp
from jax.experimental import pallas as pl

ROWS, D = 8192, 768
BLOCK_ROWS = 4096


def _copy_body(w_ref, o_ref):
    o_ref[...] = w_ref[...]


def kernel(x, W):
    del x
    return pl.pallas_call(
        _copy_body,
        grid=(ROWS // BLOCK_ROWS,),
        in_specs=[pl.BlockSpec((BLOCK_ROWS, D), lambda i: (i, 0))],
        out_specs=pl.BlockSpec((BLOCK_ROWS, D), lambda i: (i, 0)),
        out_shape=jax.ShapeDtypeStruct((ROWS, D), jnp.float32),
    )(W)
